# baseline (speedup 1.0000x reference)
import functools

import jax
import jax.numpy as jnp
from jax import lax
from jax.experimental import pallas as pl
from jax.experimental.pallas import tpu as pltpu
from jax.experimental.pallas import tpu_sc as plsc


def _logits_kernel(x_ref, wg_ref, bg_ref, out_ref):
    logits = jnp.dot(x_ref[...], wg_ref[...],
                     preferred_element_type=jnp.float32) + bg_ref[...]
    out_ref[...] = logits.T


def _gate_sc_body(logits_hbm, g_hbm, *refs, nc, ch):
    E = logits_hbm.shape[0]
    row_vs = refs[:E]
    g_v = refs[E]
    wid = lax.axis_index("s") * nc + lax.axis_index("c")
    base = wid * ch
    for e in range(E):
        pltpu.sync_copy(logits_hbm.at[e, pl.ds(base, ch)], row_vs[e])
    for j in range(ch // 16):
        sl = pl.ds(j * 16, 16)
        vals = [row_vs[e][sl] for e in range(E)]
        m1 = vals[0]
        for e in range(1, E):
            m1 = jnp.maximum(m1, vals[e])
        excluded = jnp.zeros((16,), jnp.float32)
        m2 = jnp.full((16,), -1e30, jnp.float32)
        for e in range(E):
            eqf = jnp.where(vals[e] == m1, 1.0, 0.0)
            notex = jnp.where(excluded < 0.5, 1.0, 0.0)
            is_max = eqf * notex
            excluded = excluded + is_max
            m2 = jnp.maximum(m2, vals[e] - is_max * 2e30)
        den = jnp.zeros((16,), jnp.float32)
        for e in range(E):
            den = den + jnp.exp(vals[e] - m1)
        g_v[pl.ds(j * 16, 16)] = (1.0 + jnp.exp(m2 - m1)) / den
    pltpu.sync_copy(g_v, g_hbm.at[pl.ds(base, ch)])


def _mlp_kernel(x_ref, g_ref, w1_ref, b1_ref, w2_ref, b2_ref, wv_ref,
                out0_ref, out1_ref, w1b_scr, w2b_scr):
    H = out0_ref.shape[1]

    @pl.when(pl.program_id(0) == 0)
    def _cast_weights():
        w1b_scr[...] = w1_ref[...].astype(jnp.bfloat16)
        w2b_scr[...] = w2_ref[...].astype(jnp.bfloat16)

    x = x_ref[...]
    xb = x.astype(jnp.bfloat16)
    g = g_ref[...]

    h = jnp.dot(xb, w1b_scr[...],
                preferred_element_type=jnp.float32).astype(jnp.bfloat16)
    h = jnp.maximum(h + b1_ref[...], jnp.bfloat16(0.0))
    eo = jnp.dot(h, w2b_scr[...],
                 preferred_element_type=jnp.float32) + b2_ref[...]

    wv = wv_ref[...]
    out0_ref[...] = eo[:, :H] * (g * wv[0:1, 0:1])
    out1_ref[...] = eo[:, H:] * wv[0:1, 1:2]


def kernel(vector, Wg, bg, Wf, bf, W1, b1, W2, b2, w):
    B, S, H2 = vector.shape
    H = H2 // 2
    E = Wg.shape[1]
    M = B * S
    x = vector.reshape(M, H2)

    BMG = 2048
    logits = pl.pallas_call(
        _logits_kernel,
        grid=(M // BMG,),
        in_specs=[
            pl.BlockSpec((BMG, H), lambda i: (i, 0)),
            pl.BlockSpec((H, E), lambda i: (0, 0)),
            pl.BlockSpec((1, E), lambda i: (0, 0)),
        ],
        out_specs=pl.BlockSpec((E, BMG), lambda i: (0, i)),
        out_shape=jax.ShapeDtypeStruct((E, M), jnp.float32),
        compiler_params=pltpu.CompilerParams(
            dimension_semantics=("arbitrary",),
        ),
    )(x, Wg, bg.reshape(1, E))

    info = plsc.get_sparse_core_info()
    nc, ns = info.num_cores, info.num_subcores
    ch = M // (nc * ns)
    mesh = plsc.VectorSubcoreMesh(core_axis_name="c", subcore_axis_name="s")
    gate = functools.partial(
        pl.kernel,
        mesh=mesh,
        out_type=jax.ShapeDtypeStruct((M,), jnp.float32),
        scratch_types=[pltpu.VMEM((ch,), jnp.float32) for _ in range(E + 1)],
    )(functools.partial(_gate_sc_body, nc=nc, ch=ch))
    g = gate(logits).reshape(M, 1)

    BM = 512
    b1b = b1.astype(jnp.bfloat16)
    out0, out1 = pl.pallas_call(
        _mlp_kernel,
        grid=(M // BM,),
        in_specs=[
            pl.BlockSpec((BM, H2), lambda i: (i, 0)),
            pl.BlockSpec((BM, 1), lambda i: (i, 0)),
            pl.BlockSpec((H2, H2), lambda i: (0, 0)),
            pl.BlockSpec((1, H2), lambda i: (0, 0)),
            pl.BlockSpec((H2, H2), lambda i: (0, 0)),
            pl.BlockSpec((1, H2), lambda i: (0, 0)),
            pl.BlockSpec((1, 2), lambda i: (0, 0)),
        ],
        out_specs=[
            pl.BlockSpec((BM, H), lambda i: (i, 0)),
            pl.BlockSpec((BM, H), lambda i: (i, 0)),
        ],
        out_shape=[
            jax.ShapeDtypeStruct((M, H), jnp.float32),
            jax.ShapeDtypeStruct((M, H), jnp.float32),
        ],
        scratch_shapes=[
            pltpu.VMEM((H2, H2), jnp.bfloat16),
            pltpu.VMEM((H2, H2), jnp.bfloat16),
        ],
        compiler_params=pltpu.CompilerParams(
            dimension_semantics=("arbitrary",),
        ),
    )(x, g, W1, b1b.reshape(1, H2), W2, b2.reshape(1, H2), w.reshape(1, 2))

    return (out0.reshape(B, S, H), out1.reshape(B, S, H))


# speedup vs baseline: 1.3121x; 1.3121x over previous
import jax
import jax.numpy as jnp
from jax.experimental import pallas as pl
from jax.experimental.pallas import tpu as pltpu


def _fused_kernel(x_ref, w1_ref, b1_ref, w2_ref, b2_ref, wg_ref, bg_ref,
                  wv_ref, out0_ref, out1_ref, w1b_scr, w2b_scr, wgb_scr):
    H = wg_ref.shape[0]

    @pl.when(pl.program_id(0) == 0)
    def _cast_weights():
        w1b_scr[...] = w1_ref[...].astype(jnp.bfloat16)
        w2b_scr[...] = w2_ref[...].astype(jnp.bfloat16)
        wgb_scr[...] = wg_ref[...].astype(jnp.bfloat16)

    x = x_ref[...]
    xb = x.astype(jnp.bfloat16)

    logits = jnp.dot(xb[:, :H], wgb_scr[...],
                     preferred_element_type=jnp.float32) + bg_ref[...]
    m = jnp.max(logits, axis=-1, keepdims=True)
    e = jnp.exp(logits - m)
    den = jnp.sum(e, axis=-1, keepdims=True)
    col = jax.lax.broadcasted_iota(jnp.int32, logits.shape, 1)
    am = jnp.argmax(logits, axis=-1)[:, None]
    m2 = jnp.max(jnp.where(col == am, -jnp.inf, logits),
                 axis=-1, keepdims=True)
    g = (1.0 + jnp.exp(m2 - m)) / den

    h = jnp.dot(xb, w1b_scr[...],
                preferred_element_type=jnp.float32).astype(jnp.bfloat16)
    h = jnp.maximum(h + b1_ref[...], jnp.bfloat16(0.0))
    eo = jnp.dot(h, w2b_scr[...],
                 preferred_element_type=jnp.float32) + b2_ref[...]

    wv = wv_ref[...]
    out0_ref[...] = eo[:, :H] * (g * wv[0:1, 0:1])
    out1_ref[...] = eo[:, H:] * wv[0:1, 1:2]


def kernel(vector, Wg, bg, Wf, bf, W1, b1, W2, b2, w):
    B, S, H2 = vector.shape
    H = H2 // 2
    E = Wg.shape[1]
    M = B * S
    BM = 1024
    x = vector.reshape(M, H2)

    grid = (M // BM,)
    b1b = b1.astype(jnp.bfloat16)

    out0, out1 = pl.pallas_call(
        _fused_kernel,
        grid=grid,
        in_specs=[
            pl.BlockSpec((BM, H2), lambda i: (i, 0)),
            pl.BlockSpec((H2, H2), lambda i: (0, 0)),
            pl.BlockSpec((1, H2), lambda i: (0, 0)),
            pl.BlockSpec((H2, H2), lambda i: (0, 0)),
            pl.BlockSpec((1, H2), lambda i: (0, 0)),
            pl.BlockSpec((H, E), lambda i: (0, 0)),
            pl.BlockSpec((1, E), lambda i: (0, 0)),
            pl.BlockSpec((1, 2), lambda i: (0, 0)),
        ],
        out_specs=[
            pl.BlockSpec((BM, H), lambda i: (i, 0)),
            pl.BlockSpec((BM, H), lambda i: (i, 0)),
        ],
        out_shape=[
            jax.ShapeDtypeStruct((M, H), jnp.float32),
            jax.ShapeDtypeStruct((M, H), jnp.float32),
        ],
        scratch_shapes=[
            pltpu.VMEM((H2, H2), jnp.bfloat16),
            pltpu.VMEM((H2, H2), jnp.bfloat16),
            pltpu.VMEM((H, E), jnp.bfloat16),
        ],
        compiler_params=pltpu.CompilerParams(
            dimension_semantics=("arbitrary",),
        ),
    )(x, W1, b1b.reshape(1, H2), W2, b2.reshape(1, H2),
      Wg, bg.reshape(1, E), w.reshape(1, 2))

    return (out0.reshape(B, S, H), out1.reshape(B, S, H))
